# baseline (speedup 1.0000x reference)
import jax
import jax.numpy as jnp
from jax.experimental import pallas as pl

_NUM_CLASSES = 1000
_N = 1024


_BC = 200


def _body(x_ref, o_ref, *, bc):
    c = pl.program_id(1) * bc
    row = c + jax.lax.broadcasted_iota(jnp.int32, (1, bc, _N), 1)
    o_ref[...] = (row == x_ref[...]).astype(jnp.float32)


def kernel(x):
    import functools
    xt = x.astype(jnp.int32).T.reshape(26, 1, _N)
    yt = pl.pallas_call(
        functools.partial(_body, bc=_BC),
        grid=(26, _NUM_CLASSES // _BC),
        in_specs=[pl.BlockSpec((1, 1, _N), lambda j, c: (j, 0, 0))],
        out_specs=pl.BlockSpec((1, _BC, _N), lambda j, c: (j, c, 0)),
        out_shape=jax.ShapeDtypeStruct((26, _NUM_CLASSES, _N), jnp.float32),
    )(xt)
    return jnp.transpose(yt, (2, 0, 1))


# speedup vs baseline: 2.0866x; 2.0866x over previous
import jax
import jax.numpy as jnp
from jax.experimental import pallas as pl

_NUM_CLASSES = 1000
_N = 1024


def _body(x_ref, o_ref):
    row = jax.lax.broadcasted_iota(jnp.int32, (1, _NUM_CLASSES, _N), 1)
    o_ref[...] = (row == x_ref[...]).astype(jnp.float32)


def kernel(x):
    xt = x.astype(jnp.int32).T.reshape(26, 1, _N)
    yt = pl.pallas_call(
        _body,
        grid=(26,),
        in_specs=[pl.BlockSpec((1, 1, _N), lambda j: (j, 0, 0))],
        out_specs=pl.BlockSpec((1, _NUM_CLASSES, _N), lambda j: (j, 0, 0)),
        out_shape=jax.ShapeDtypeStruct((26, _NUM_CLASSES, _N), jnp.float32),
    )(xt)
    return jnp.transpose(yt, (2, 0, 1))


# speedup vs baseline: 2.1949x; 1.0519x over previous
import jax
import jax.numpy as jnp
from jax.experimental import pallas as pl

_NUM_CLASSES = 1000
_N = 1024


def _body(x_ref, o_ref):
    j = pl.program_id(0)
    xv = x_ref[pl.ds(j, 1), :].reshape(1, 1, _N)
    row = jax.lax.broadcasted_iota(jnp.int32, (1, _NUM_CLASSES, _N), 1)
    o_ref[...] = (row == xv).astype(jnp.float32)


def kernel(x):
    xt = x.astype(jnp.int32).T
    yt = pl.pallas_call(
        _body,
        grid=(26,),
        in_specs=[pl.BlockSpec((26, _N), lambda j: (0, 0))],
        out_specs=pl.BlockSpec((1, _NUM_CLASSES, _N), lambda j: (j, 0, 0)),
        out_shape=jax.ShapeDtypeStruct((26, _NUM_CLASSES, _N), jnp.float32),
    )(xt)
    return jnp.transpose(yt, (2, 0, 1))
